# baseline (speedup 1.0000x reference)
import functools

import jax
import jax.numpy as jnp
from jax import lax
from jax.experimental import pallas as pl
from jax.experimental.pallas import tpu as pltpu
from jax.experimental.pallas import tpu_sc as plsc

_K = 1024
_EPS_LN = 1e-5

_NC, _NS, _NL = 2, 16, 16
_NW = _NC * _NS

_HI = jax.lax.Precision.DEFAULT


def _ln(y, g, b):
    m = jnp.mean(y, axis=-1, keepdims=True)
    yc = y - m
    v = jnp.mean(yc * yc, axis=-1, keepdims=True)
    return yc * jax.lax.rsqrt(v + _EPS_LN) * g + b


def _fused_body(x_ref, W1_ref, b1_ref, g1_ref, bb1_ref, pat_ref, patx_ref,
                W2_ref, b2_ref, g2_ref, bb2_ref,
                emb_ref, logit_ref, idx_ref, q_ref):
    x = x_ref[...]
    W1 = W1_ref[...]
    q = jax.lax.dot_general(x, W1, (((1,), (0,)), ((), ())),
                            precision=_HI, preferred_element_type=jnp.float32)
    q = q + b1_ref[...]
    q = _ln(q, g1_ref[...], bb1_ref[...])
    q_ref[...] = q

    qn = q / jnp.maximum(
        jnp.sqrt(jnp.sum(q * q, axis=-1, keepdims=True)), 1e-12)

    pat = pat_ref[...]
    kn = pat / jnp.maximum(
        jnp.sqrt(jnp.sum(pat * pat, axis=-1, keepdims=True)), 1e-12)

    logits = jax.lax.dot_general(
        qn, kn, (((1,), (1,)), ((), ())),
        precision=_HI, preferred_element_type=jnp.float32) * 0.5
    logit_ref[...] = logits

    m = jnp.max(logits, axis=-1, keepdims=True)
    asg = (logits == m).astype(jnp.float32)

    lowx = jax.lax.dot_general(asg, patx_ref[...], (((1,), (0,)), ((), ())),
                               precision=_HI, preferred_element_type=jnp.float32)
    dz = pat.shape[1]
    low = lowx[:, :dz]
    idx_f = lowx[:, dz:dz + 1] * 16.0 + lowx[:, dz + 1:dz + 2]
    idx_ref[...] = idx_f.astype(jnp.int32)

    y = jax.lax.dot_general(low, W2_ref[...], (((1,), (0,)), ((), ())),
                            precision=_HI, preferred_element_type=jnp.float32)
    y = y + b2_ref[...]
    emb_ref[...] = _ln(y, g2_ref[...], bb2_ref[...])


def _sc_mesh():
    return plsc.VectorSubcoreMesh(core_axis_name="c", subcore_axis_name="s",
                                  num_cores=_NC, num_subcores=_NS)


def _worker_id():
    return lax.axis_index("c") * _NS + lax.axis_index("s")


def _make_sc_zeros(nk):
    per = nk // _NW
    zw = 65536
    while per % zw:
        zw //= 2
    assert zw >= _NL

    @functools.partial(
        pl.kernel, mesh=_sc_mesh(),
        out_type=jax.ShapeDtypeStruct((nk,), jnp.float32),
        scratch_types=[pltpu.VMEM((zw,), jnp.float32)],
    )
    def sc_zeros(out_hbm, zbuf):
        @pl.loop(0, zw // _NL)
        def _fill(i):
            zbuf[pl.ds(i * _NL, _NL)] = jnp.zeros((_NL,), jnp.float32)

        base = _worker_id() * per

        @pl.loop(0, per // zw)
        def _store(j):
            pltpu.sync_copy(zbuf, out_hbm.at[pl.ds(base + j * zw, zw)])

    return sc_zeros


def _make_sc_scatter(n, k):
    per = n // _NW
    assert per % 128 == 0

    @functools.partial(
        pl.kernel, mesh=_sc_mesh(), out_type=(),
        scratch_types=[
            pltpu.VMEM((per,), jnp.int32),
            pltpu.VMEM((per,), jnp.int32),
            pltpu.VMEM((per,), jnp.float32),
            pltpu.SemaphoreType.DMA,
        ],
    )
    def sc_scatter(idx_hbm, asg_hbm, idxbuf, posbuf, onesbuf, sem):
        base = _worker_id() * per
        pltpu.sync_copy(idx_hbm.at[pl.ds(base, per)], idxbuf)

        @pl.loop(0, per // _NL)
        def _pos(j):
            iv = idxbuf[pl.ds(j * _NL, _NL)]
            iv = jnp.minimum(iv, k - 1)
            rows = (base + j * _NL) + lax.iota(jnp.int32, _NL)
            posbuf[pl.ds(j * _NL, _NL)] = rows * k + iv
            onesbuf[pl.ds(j * _NL, _NL)] = jnp.ones((_NL,), jnp.float32)

        @pl.loop(0, per // 128)
        def _scat(c):
            pltpu.async_copy(
                onesbuf.at[pl.ds(c * 128, 128)],
                asg_hbm.at[posbuf.at[pl.ds(c * 128, 128)]],
                sem).wait()

    return sc_scatter


def kernel(x, W1, b1, ln1_g, ln1_b, patterns, W2, b2, ln2_g, ln2_b):
    B, T, D = x.shape
    Dz = W1.shape[1]
    K = patterns.shape[0]
    N = B * T
    TN = 1536
    grid = (N // TN,)

    xf = x.reshape(N, D)
    b1r = b1.reshape(1, Dz)
    g1r = ln1_g.reshape(1, Dz)
    bb1r = ln1_b.reshape(1, Dz)
    b2r = b2.reshape(1, D)
    g2r = ln2_g.reshape(1, D)
    bb2r = ln2_b.reshape(1, D)
    ki = jnp.arange(K, dtype=jnp.int32)
    patx = jnp.concatenate(
        [patterns, (ki // 16).astype(jnp.float32)[:, None],
         (ki % 16).astype(jnp.float32)[:, None]], axis=1)

    full = lambda shape: pl.BlockSpec(shape, lambda i: (0, 0))
    out = pl.pallas_call(
        _fused_body,
        grid=grid,
        in_specs=[
            pl.BlockSpec((TN, D), lambda i: (i, 0)),
            full((D, Dz)),
            full((1, Dz)), full((1, Dz)), full((1, Dz)),
            full((K, Dz)),
            full((K, Dz + 2)),
            full((Dz, D)),
            full((1, D)), full((1, D)), full((1, D)),
        ],
        out_specs=[
            pl.BlockSpec((TN, D), lambda i: (i, 0)),
            pl.BlockSpec((TN, K), lambda i: (i, 0)),
            pl.BlockSpec((TN, 1), lambda i: (i, 0)),
            pl.BlockSpec((TN, Dz), lambda i: (i, 0)),
        ],
        out_shape=[
            jax.ShapeDtypeStruct((N, D), jnp.float32),
            jax.ShapeDtypeStruct((N, K), jnp.float32),
            jax.ShapeDtypeStruct((N, 1), jnp.int32),
            jax.ShapeDtypeStruct((N, Dz), jnp.float32),
        ],
    )(xf, W1, b1r, g1r, bb1r, patterns, patx, W2, b2r, g2r, bb2r)

    emb, logits, idx, q = out

    asg_flat = _make_sc_zeros(N * K)()
    asg_ref = jax.new_ref(asg_flat)
    _make_sc_scatter(N, K)(idx.reshape(N), asg_ref)
    asg = asg_ref[...].reshape(N, K)

    return (emb.reshape(B, T, D), asg.reshape(B, T, K),
            logits.reshape(B, T, K), idx.reshape(B, T), q.reshape(B, T, Dz))


# speedup vs baseline: 1.9091x; 1.9091x over previous
import functools

import jax
import jax.numpy as jnp
from jax import lax
from jax.experimental import pallas as pl
from jax.experimental.pallas import tpu as pltpu
from jax.experimental.pallas import tpu_sc as plsc

_K = 1024
_EPS_LN = 1e-5

_NC, _NS, _NL = 2, 16, 16
_NW = _NC * _NS

_HI = jax.lax.Precision.DEFAULT


def _ln(y, g, b):
    m = jnp.mean(y, axis=-1, keepdims=True)
    yc = y - m
    v = jnp.mean(yc * yc, axis=-1, keepdims=True)
    return yc * jax.lax.rsqrt(v + _EPS_LN) * g + b


def _fused_body(x_ref, W1_ref, b1_ref, g1_ref, bb1_ref, pat_ref, patx_ref,
                W2_ref, b2_ref, g2_ref, bb2_ref,
                emb_ref, asg_ref, logit_ref, idx_ref, q_ref):
    x = x_ref[...]
    W1 = W1_ref[...]
    q = jax.lax.dot_general(x, W1, (((1,), (0,)), ((), ())),
                            precision=_HI, preferred_element_type=jnp.float32)
    q = q + b1_ref[...]
    q = _ln(q, g1_ref[...], bb1_ref[...])
    q_ref[...] = q

    qn = q / jnp.maximum(
        jnp.sqrt(jnp.sum(q * q, axis=-1, keepdims=True)), 1e-12)

    pat = pat_ref[...]
    kn = pat / jnp.maximum(
        jnp.sqrt(jnp.sum(pat * pat, axis=-1, keepdims=True)), 1e-12)

    logits = jax.lax.dot_general(
        qn, kn, (((1,), (1,)), ((), ())),
        precision=_HI, preferred_element_type=jnp.float32) * 0.5
    logit_ref[...] = logits

    m = jnp.max(logits, axis=-1, keepdims=True)
    asg = (logits == m).astype(jnp.float32)
    asg_ref[...] = asg

    lowx = jax.lax.dot_general(asg, patx_ref[...], (((1,), (0,)), ((), ())),
                               precision=_HI, preferred_element_type=jnp.float32)
    dz = pat.shape[1]
    low = lowx[:, :dz]
    idx_f = lowx[:, dz:dz + 1] * 16.0 + lowx[:, dz + 1:dz + 2]
    idx_ref[...] = idx_f.astype(jnp.int32).reshape(1, 1, idx_f.shape[0])

    y = jax.lax.dot_general(low, W2_ref[...], (((1,), (0,)), ((), ())),
                            precision=_HI, preferred_element_type=jnp.float32)
    y = y + b2_ref[...]
    emb_ref[...] = _ln(y, g2_ref[...], bb2_ref[...])


def _sc_mesh():
    return plsc.VectorSubcoreMesh(core_axis_name="c", subcore_axis_name="s",
                                  num_cores=_NC, num_subcores=_NS)


def _worker_id():
    return lax.axis_index("c") * _NS + lax.axis_index("s")


def _make_sc_zeros(nk):
    per = nk // _NW
    zw = 65536
    while per % zw:
        zw //= 2
    assert zw >= _NL

    @functools.partial(
        pl.kernel, mesh=_sc_mesh(),
        out_type=jax.ShapeDtypeStruct((nk,), jnp.float32),
        scratch_types=[pltpu.VMEM((zw,), jnp.float32)],
    )
    def sc_zeros(out_hbm, zbuf):
        @pl.loop(0, zw // _NL)
        def _fill(i):
            zbuf[pl.ds(i * _NL, _NL)] = jnp.zeros((_NL,), jnp.float32)

        base = _worker_id() * per

        @pl.loop(0, per // zw)
        def _store(j):
            pltpu.sync_copy(zbuf, out_hbm.at[pl.ds(base + j * zw, zw)])

    return sc_zeros


def _make_sc_scatter(n, k):
    per = n // _NW
    assert per % 128 == 0

    @functools.partial(
        pl.kernel, mesh=_sc_mesh(), out_type=(),
        scratch_types=[
            pltpu.VMEM((per,), jnp.int32),
            pltpu.VMEM((per,), jnp.int32),
            pltpu.VMEM((per,), jnp.float32),
            pltpu.SemaphoreType.DMA,
        ],
    )
    def sc_scatter(idx_hbm, asg_hbm, idxbuf, posbuf, onesbuf, sem):
        base = _worker_id() * per
        pltpu.sync_copy(idx_hbm.at[pl.ds(base, per)], idxbuf)

        @pl.loop(0, per // _NL)
        def _pos(j):
            iv = idxbuf[pl.ds(j * _NL, _NL)]
            iv = jnp.minimum(iv, k - 1)
            rows = (base + j * _NL) + lax.iota(jnp.int32, _NL)
            posbuf[pl.ds(j * _NL, _NL)] = rows * k + iv
            onesbuf[pl.ds(j * _NL, _NL)] = jnp.ones((_NL,), jnp.float32)

        @pl.loop(0, per // 128)
        def _scat(c):
            pltpu.async_copy(
                onesbuf.at[pl.ds(c * 128, 128)],
                asg_hbm.at[posbuf.at[pl.ds(c * 128, 128)]],
                sem).wait()

    return sc_scatter


def kernel(x, W1, b1, ln1_g, ln1_b, patterns, W2, b2, ln2_g, ln2_b):
    B, T, D = x.shape
    Dz = W1.shape[1]
    K = patterns.shape[0]
    N = B * T
    TN = 1536
    grid = (N // TN,)

    xf = x.reshape(N, D)
    b1r = b1.reshape(1, Dz)
    g1r = ln1_g.reshape(1, Dz)
    bb1r = ln1_b.reshape(1, Dz)
    b2r = b2.reshape(1, D)
    g2r = ln2_g.reshape(1, D)
    bb2r = ln2_b.reshape(1, D)
    ki = jnp.arange(K, dtype=jnp.int32)
    patx = jnp.concatenate(
        [patterns, (ki // 16).astype(jnp.float32)[:, None],
         (ki % 16).astype(jnp.float32)[:, None]], axis=1)

    full = lambda shape: pl.BlockSpec(shape, lambda i: (0, 0))
    out = pl.pallas_call(
        _fused_body,
        grid=grid,
        in_specs=[
            pl.BlockSpec((TN, D), lambda i: (i, 0)),
            full((D, Dz)),
            full((1, Dz)), full((1, Dz)), full((1, Dz)),
            full((K, Dz)),
            full((K, Dz + 2)),
            full((Dz, D)),
            full((1, D)), full((1, D)), full((1, D)),
        ],
        out_specs=[
            pl.BlockSpec((TN, D), lambda i: (i, 0)),
            pl.BlockSpec((TN, K), lambda i: (i, 0)),
            pl.BlockSpec((TN, K), lambda i: (i, 0)),
            pl.BlockSpec((1, 1, TN), lambda i: (i, 0, 0)),
            pl.BlockSpec((TN, Dz), lambda i: (i, 0)),
        ],
        out_shape=[
            jax.ShapeDtypeStruct((N, D), jnp.float32),
            jax.ShapeDtypeStruct((N, K), jnp.float32),
            jax.ShapeDtypeStruct((N, K), jnp.float32),
            jax.ShapeDtypeStruct((N // TN, 1, TN), jnp.int32),
            jax.ShapeDtypeStruct((N, Dz), jnp.float32),
        ],
    )(xf, W1, b1r, g1r, bb1r, patterns, patx, W2, b2r, g2r, bb2r)

    emb, asg, logits, idx, q = out

    return (emb.reshape(B, T, D), asg.reshape(B, T, K),
            logits.reshape(B, T, K), idx.reshape(B, T), q.reshape(B, T, Dz))


# speedup vs baseline: 1.9283x; 1.0101x over previous
import jax
import jax.numpy as jnp
from jax.experimental import pallas as pl

_EPS_LN = 1e-5
_PREC = jax.lax.Precision.DEFAULT


def _ln(y, g, b):
    m = jnp.mean(y, axis=-1, keepdims=True)
    yc = y - m
    v = jnp.mean(yc * yc, axis=-1, keepdims=True)
    return yc * jax.lax.rsqrt(v + _EPS_LN) * g + b


def _fused_body(x_ref, W1_ref, b1_ref, g1_ref, bb1_ref, pat_ref, patx_ref,
                W2_ref, b2_ref, g2_ref, bb2_ref,
                emb_ref, asg_ref, logit_ref, idx_ref, q_ref):
    x = x_ref[...]
    q = jax.lax.dot_general(x, W1_ref[...], (((1,), (0,)), ((), ())),
                            precision=_PREC,
                            preferred_element_type=jnp.float32)
    q = q + b1_ref[...]
    q = _ln(q, g1_ref[...], bb1_ref[...])
    q_ref[...] = q

    qn = q / jnp.maximum(
        jnp.sqrt(jnp.sum(q * q, axis=-1, keepdims=True)), 1e-12)

    pat = pat_ref[...]
    kn = pat / jnp.maximum(
        jnp.sqrt(jnp.sum(pat * pat, axis=-1, keepdims=True)), 1e-12)

    logits = jax.lax.dot_general(
        qn, kn, (((1,), (1,)), ((), ())),
        precision=_PREC, preferred_element_type=jnp.float32) * 0.5
    logit_ref[...] = logits

    m = jnp.max(logits, axis=-1, keepdims=True)
    asg = (logits == m).astype(jnp.float32)
    asg_ref[...] = asg

    lowx = jax.lax.dot_general(asg, patx_ref[...], (((1,), (0,)), ((), ())),
                               precision=_PREC,
                               preferred_element_type=jnp.float32)
    dz = pat.shape[1]
    low = lowx[:, :dz]
    idx_f = lowx[:, dz:dz + 1] * 16.0 + lowx[:, dz + 1:dz + 2]
    idx_ref[...] = idx_f.astype(jnp.int32)

    y = jax.lax.dot_general(low, W2_ref[...], (((1,), (0,)), ((), ())),
                            precision=_PREC,
                            preferred_element_type=jnp.float32)
    y = y + b2_ref[...]
    emb_ref[...] = _ln(y, g2_ref[...], bb2_ref[...])


def kernel(x, W1, b1, ln1_g, ln1_b, patterns, W2, b2, ln2_g, ln2_b):
    B, T, D = x.shape
    Dz = W1.shape[1]
    K = patterns.shape[0]
    N = B * T
    TN = 1536
    grid = (N // TN,)

    xf = x.reshape(N, D)
    b1r = b1.reshape(1, Dz)
    g1r = ln1_g.reshape(1, Dz)
    bb1r = ln1_b.reshape(1, Dz)
    b2r = b2.reshape(1, D)
    g2r = ln2_g.reshape(1, D)
    bb2r = ln2_b.reshape(1, D)
    ki = jnp.arange(K, dtype=jnp.int32)
    patx = jnp.concatenate(
        [patterns, (ki // 16).astype(jnp.float32)[:, None],
         (ki % 16).astype(jnp.float32)[:, None]], axis=1)

    full = lambda shape: pl.BlockSpec(shape, lambda i: (0, 0))
    out = pl.pallas_call(
        _fused_body,
        grid=grid,
        in_specs=[
            pl.BlockSpec((TN, D), lambda i: (i, 0)),
            full((D, Dz)),
            full((1, Dz)), full((1, Dz)), full((1, Dz)),
            full((K, Dz)),
            full((K, Dz + 2)),
            full((Dz, D)),
            full((1, D)), full((1, D)), full((1, D)),
        ],
        out_specs=[
            pl.BlockSpec((TN, D), lambda i: (i, 0)),
            pl.BlockSpec((TN, K), lambda i: (i, 0)),
            pl.BlockSpec((TN, K), lambda i: (i, 0)),
            pl.BlockSpec((TN, 1), lambda i: (i, 0)),
            pl.BlockSpec((TN, Dz), lambda i: (i, 0)),
        ],
        out_shape=[
            jax.ShapeDtypeStruct((N, D), jnp.float32),
            jax.ShapeDtypeStruct((N, K), jnp.float32),
            jax.ShapeDtypeStruct((N, K), jnp.float32),
            jax.ShapeDtypeStruct((N, 1), jnp.int32),
            jax.ShapeDtypeStruct((N, Dz), jnp.float32),
        ],
    )(xf, W1, b1r, g1r, bb1r, patterns, patx, W2, b2r, g2r, bb2r)

    emb, asg, logits, idx, q = out
    return (emb.reshape(B, T, D), asg.reshape(B, T, K),
            logits.reshape(B, T, K), idx.reshape(B, T), q.reshape(B, T, Dz))
